# Initial kernel scaffold; baseline (speedup 1.0000x reference)
import jax
import jax.numpy as jnp
from jax.experimental import pallas as pl


def kernel(h, e, edge_index, pos_enc, atom_emb, bond_emb, Wp, bp, Wpe, bpe, Wq, Wk, Wv, We, Wo, W1, b1, W2, b2, Ue, Wr1, br1, Wr2, br2, Wr3, br3):
    raise NotImplementedError("write your pallas kernel here")


# speedup vs baseline: 1.2021x; 1.2021x over previous
import functools

import jax
import jax.numpy as jnp
import numpy as np
from jax.experimental import pallas as pl
from jax.experimental.pallas import tpu as pltpu

N = 10000
E = 320000
D = 128
H = 8
DH = 16
L = 3
PE_DIM = 16
ATOM_VOCAB = 100
BOND_VOCAB = 10
NAF = 9
NBF = 3

BN = 1000
BE = 8000

_INV_SQRT_DH = 0.25


def _ln(x):
    m = jnp.mean(x, axis=-1, keepdims=True)
    v = jnp.mean((x - m) ** 2, axis=-1, keepdims=True)
    return (x - m) * jax.lax.rsqrt(v + 1e-5)


def _dot(a, b):
    return jnp.dot(a, b, preferred_element_type=jnp.float32)


def _node_enc_body(h_ref, pos_ref, aemb_ref, Wp_ref, Wpe_ref, bpe_ref, out_ref):
    iota = jax.lax.broadcasted_iota(jnp.int32, (BN, 1024), 1)
    oh = jnp.zeros((BN, 1024), jnp.float32)
    for c in range(NAF):
        idx = h_ref[:, c][:, None] + c * ATOM_VOCAB
        oh = oh + (iota == idx).astype(jnp.float32)
    acc = _dot(oh, aemb_ref[...])
    p = _dot(pos_ref[...], Wp_ref[...])
    p = jnp.tanh(_dot(p, Wpe_ref[...]) + bpe_ref[...])
    out_ref[...] = acc + p


def _node_enc(h, pos_enc, aemb_pad, Wp, Wpe, bpe):
    return pl.pallas_call(
        _node_enc_body,
        grid=(N // BN,),
        in_specs=[
            pl.BlockSpec((BN, NAF), lambda i: (i, 0)),
            pl.BlockSpec((BN, PE_DIM), lambda i: (i, 0)),
            pl.BlockSpec((1024, D), lambda i: (0, 0)),
            pl.BlockSpec((PE_DIM, D), lambda i: (0, 0)),
            pl.BlockSpec((D, D), lambda i: (0, 0)),
            pl.BlockSpec((1, D), lambda i: (0, 0)),
        ],
        out_specs=pl.BlockSpec((BN, D), lambda i: (i, 0)),
        out_shape=jax.ShapeDtypeStruct((N, D), jnp.float32),
    )(h, pos_enc, aemb_pad, Wp, Wpe, bpe)


def _bond_enc_body(e_ref, bemb_ref, We0_ref, ee_ref, ep_ref):
    iota = jax.lax.broadcasted_iota(jnp.int32, (BE, 32), 1)
    oh = jnp.zeros((BE, 32), jnp.float32)
    for c in range(NBF):
        idx = e_ref[:, c][:, None] + c * BOND_VOCAB
        oh = oh + (iota == idx).astype(jnp.float32)
    ee = _dot(oh, bemb_ref[...])
    ee_ref[...] = ee
    ep_ref[...] = _dot(ee, We0_ref[...])


def _bond_enc(e, bemb_pad, We0):
    return pl.pallas_call(
        _bond_enc_body,
        grid=(E // BE,),
        in_specs=[
            pl.BlockSpec((BE, NBF), lambda i: (i, 0)),
            pl.BlockSpec((32, D), lambda i: (0, 0)),
            pl.BlockSpec((D, D), lambda i: (0, 0)),
        ],
        out_specs=[
            pl.BlockSpec((BE, D), lambda i: (i, 0)),
            pl.BlockSpec((BE, D), lambda i: (i, 0)),
        ],
        out_shape=[
            jax.ShapeDtypeStruct((E, D), jnp.float32),
            jax.ShapeDtypeStruct((E, D), jnp.float32),
        ],
    )(e, bemb_pad, We0)


def _qkv_body(hn_ref, Wq_ref, Wkv_ref, q_ref, kv_ref):
    q_ref[...] = _dot(hn_ref[...], Wq_ref[...])
    kv_ref[...] = _dot(hn_ref[...], Wkv_ref[...])


def _qkv(hn, Wq, Wkv):
    return pl.pallas_call(
        _qkv_body,
        grid=(N // BN,),
        in_specs=[
            pl.BlockSpec((BN, D), lambda i: (i, 0)),
            pl.BlockSpec((D, D), lambda i: (0, 0)),
            pl.BlockSpec((D, 2 * D), lambda i: (0, 0)),
        ],
        out_specs=[
            pl.BlockSpec((BN, D), lambda i: (i, 0)),
            pl.BlockSpec((BN, 2 * D), lambda i: (i, 0)),
        ],
        out_shape=[
            jax.ShapeDtypeStruct((N, D), jnp.float32),
            jax.ShapeDtypeStruct((N, 2 * D), jnp.float32),
        ],
    )(hn, Wq, Wkv)


def _node_upd_body(hn_ref, agg_ref, Wo_ref, W1_ref, b1_ref, W2_ref, b2_ref, out_ref):
    x = hn_ref[...] + _dot(agg_ref[...], Wo_ref[...])
    hn2 = _ln(x)
    ffn = _dot(jnp.maximum(_dot(hn2, W1_ref[...]) + b1_ref[...], 0.0), W2_ref[...]) + b2_ref[...]
    out_ref[...] = _ln(hn2 + ffn)


def _node_upd(hn, agg, Wo, W1, b1, W2, b2):
    return pl.pallas_call(
        _node_upd_body,
        grid=(N // BN,),
        in_specs=[
            pl.BlockSpec((BN, D), lambda i: (i, 0)),
            pl.BlockSpec((BN, D), lambda i: (i, 0)),
            pl.BlockSpec((D, D), lambda i: (0, 0)),
            pl.BlockSpec((D, 2 * D), lambda i: (0, 0)),
            pl.BlockSpec((1, 2 * D), lambda i: (0, 0)),
            pl.BlockSpec((2 * D, D), lambda i: (0, 0)),
            pl.BlockSpec((1, D), lambda i: (0, 0)),
        ],
        out_specs=pl.BlockSpec((BN, D), lambda i: (i, 0)),
        out_shape=jax.ShapeDtypeStruct((N, D), jnp.float32),
    )(hn, agg, Wo, W1, b1, W2, b2)


def _ee_upd_body(ee_ref, ep_ref, Ue_ref, Wen_ref, eeo_ref, epo_ref):
    t = _ln(ee_ref[...] + jnp.maximum(_dot(ep_ref[...], Ue_ref[...]), 0.0))
    eeo_ref[...] = t
    epo_ref[...] = _dot(t, Wen_ref[...])


def _ee_upd(ee, ep, Ue, Wen):
    return pl.pallas_call(
        _ee_upd_body,
        grid=(E // BE,),
        in_specs=[
            pl.BlockSpec((BE, D), lambda i: (i, 0)),
            pl.BlockSpec((BE, D), lambda i: (i, 0)),
            pl.BlockSpec((D, D), lambda i: (0, 0)),
            pl.BlockSpec((D, D), lambda i: (0, 0)),
        ],
        out_specs=[
            pl.BlockSpec((BE, D), lambda i: (i, 0)),
            pl.BlockSpec((BE, D), lambda i: (i, 0)),
        ],
        out_shape=[
            jax.ShapeDtypeStruct((E, D), jnp.float32),
            jax.ShapeDtypeStruct((E, D), jnp.float32),
        ],
    )(ee, ep, Ue, Wen)


def _readout_body(hn_ref, Wr1_ref, br1_ref, Wr2_ref, br2_ref, Wr3t_ref, br3_ref,
                  out_ref, acc_ref):
    i = pl.program_id(0)

    @pl.when(i == 0)
    def _():
        acc_ref[...] = jnp.zeros_like(acc_ref)

    acc_ref[...] += jnp.sum(hn_ref[...], axis=0, keepdims=True)

    @pl.when(i == pl.num_programs(0) - 1)
    def _():
        hg = acc_ref[...] * (1.0 / N)
        o = jnp.maximum(_dot(hg, Wr1_ref[...]) + br1_ref[...], 0.0)
        o = jnp.maximum(_dot(o, Wr2_ref[...]) + br2_ref[...], 0.0)
        o = jnp.sum(o * Wr3t_ref[...], axis=-1, keepdims=True) + br3_ref[...]
        out_ref[...] = jax.nn.sigmoid(o)


def _readout(hn, Wr1, br1, Wr2, br2, Wr3t, br3):
    return pl.pallas_call(
        _readout_body,
        grid=(N // BN,),
        in_specs=[
            pl.BlockSpec((BN, D), lambda i: (i, 0)),
            pl.BlockSpec((D, D // 2), lambda i: (0, 0)),
            pl.BlockSpec((1, D // 2), lambda i: (0, 0)),
            pl.BlockSpec((D // 2, D // 4), lambda i: (0, 0)),
            pl.BlockSpec((1, D // 4), lambda i: (0, 0)),
            pl.BlockSpec((1, D // 4), lambda i: (0, 0)),
            pl.BlockSpec((1, 1), lambda i: (0, 0)),
        ],
        out_specs=pl.BlockSpec((1, 1), lambda i: (0, 0)),
        out_shape=jax.ShapeDtypeStruct((1, 1), jnp.float32),
        scratch_shapes=[pltpu.VMEM((1, D), jnp.float32)],
    )(hn, Wr1, br1, Wr2, br2, Wr3t, br3)


def _edge_stage(q, kv, ep, src, dst):
    k = kv[:, :D]
    v = kv[:, D:]
    qd = q[dst].reshape(E, H, DH)
    ks = k[src].reshape(E, H, DH)
    vs = v[src].reshape(E, H, DH)
    s = jnp.sum(qd * ks * ep.reshape(E, H, DH), axis=-1) * _INV_SQRT_DH
    ex = jnp.exp(jnp.clip(s, -5.0, 5.0))
    den = jax.ops.segment_sum(ex, dst, num_segments=N)
    agg = jax.ops.segment_sum(ex[:, :, None] * vs, dst, num_segments=N).reshape(N, D)
    return agg / jnp.repeat(den + 1e-9, DH, axis=1)


def kernel(h, e, edge_index, pos_enc, atom_emb, bond_emb, Wp, bp, Wpe, bpe,
           Wq, Wk, Wv, We, Wo, W1, b1, W2, b2, Ue, Wr1, br1, Wr2, br2, Wr3, br3):
    src = edge_index[0]
    dst = edge_index[1]

    aemb_pad = jnp.pad(atom_emb, ((0, 1024 - NAF * ATOM_VOCAB), (0, 0)))
    bemb_pad = jnp.pad(bond_emb, ((0, 32 - NBF * BOND_VOCAB), (0, 0)))
    bpe2 = (bpe + bp @ Wpe).reshape(1, D)
    b1r = b1.reshape(L, 1, 2 * D)
    b2r = b2.reshape(L, 1, D)

    hn = _node_enc(h, pos_enc, aemb_pad, Wp, Wpe, bpe2)
    ee, ep = _bond_enc(e, bemb_pad, We[0])

    for l in range(L):
        Wkv = jnp.concatenate([Wk[l], Wv[l]], axis=1)
        q, kv = _qkv(hn, Wq[l], Wkv)
        agg = _edge_stage(q, kv, ep, src, dst)
        hn = _node_upd(hn, agg, Wo[l], W1[l], b1r[l], W2[l], b2r[l])
        if l + 1 < L:
            ee, ep = _ee_upd(ee, ep, Ue[l], We[l + 1])

    return _readout(hn, Wr1, br1.reshape(1, D // 2), Wr2, br2.reshape(1, D // 4),
                    Wr3.reshape(1, D // 4), br3.reshape(1, 1))
